# baseline (device time: 114056 ns/iter reference)
import jax
import jax.numpy as jnp
from jax import lax
from jax.experimental import pallas as pl
from jax.experimental.pallas import tpu as pltpu

N_DEV = 8
M = 1024
N = 1024
CH = M // N_DEV


def _gelu(z):
    return 0.5 * z * (1.0 + jnp.tanh(0.7978845608 * (z + 0.044715 * z * z * z)))


def kernel(A, B):
    def body(
        a_ref,
        b_ref,
        out_ref,
        z_ref,
        rs_recv,
        rs_send,
        rs_send_sems,
        rs_recv_sems,
        ag_send_sems,
        ag_recv_sems,
    ):
        my = lax.axis_index("i")
        left = (my - 1) % N_DEV
        right = (my + 1) % N_DEV

        barrier = pltpu.get_barrier_semaphore()
        for nbr in (left, right):
            pl.semaphore_signal(
                barrier, inc=1, device_id=(nbr,), device_id_type=pl.DeviceIdType.MESH
            )
        pl.semaphore_wait(barrier, 2)

        z_ref[:, :] = jnp.dot(
            a_ref[:, :], b_ref[:, :], preferred_element_type=jnp.float32
        )

        def rows(c):
            return pl.ds(c * CH, CH)

        for s in range(N_DEV - 1):
            c_send = (my - s - 1) % N_DEV
            slot = s % 2
            if s == 0:
                rs_send[slot, :, :] = z_ref[rows(c_send), :]
            else:
                rs_send[slot, :, :] = rs_recv[s - 1] + z_ref[rows(c_send), :]
            rdma = pltpu.make_async_remote_copy(
                src_ref=rs_send.at[slot],
                dst_ref=rs_recv.at[s],
                send_sem=rs_send_sems.at[s],
                recv_sem=rs_recv_sems.at[s],
                device_id=(right,),
                device_id_type=pl.DeviceIdType.MESH,
            )
            rdma.start()
            rdma.wait()

        zred = rs_recv[N_DEV - 2] + z_ref[rows(my), :]
        out_ref[rows(my), :] = _gelu(zred)

        for t in range(N_DEV - 1):
            c_send = (my - t) % N_DEV
            rdma = pltpu.make_async_remote_copy(
                src_ref=out_ref.at[rows(c_send)],
                dst_ref=out_ref.at[rows(c_send)],
                send_sem=ag_send_sems.at[t],
                recv_sem=ag_recv_sems.at[t],
                device_id=(right,),
                device_id_type=pl.DeviceIdType.MESH,
            )
            rdma.start()
            rdma.wait()

    return pl.pallas_call(
        body,
        out_shape=jax.ShapeDtypeStruct((M, N), jnp.float32),
        in_specs=[
            pl.BlockSpec(memory_space=pltpu.VMEM),
            pl.BlockSpec(memory_space=pltpu.VMEM),
        ],
        out_specs=pl.BlockSpec(memory_space=pltpu.VMEM),
        scratch_shapes=[
            pltpu.VMEM((M, N), jnp.float32),
            pltpu.VMEM((N_DEV - 1, CH, N), jnp.float32),
            pltpu.VMEM((2, CH, N), jnp.float32),
            pltpu.SemaphoreType.DMA((N_DEV - 1,)),
            pltpu.SemaphoreType.DMA((N_DEV - 1,)),
            pltpu.SemaphoreType.DMA((N_DEV - 1,)),
            pltpu.SemaphoreType.DMA((N_DEV - 1,)),
        ],
        compiler_params=pltpu.CompilerParams(collective_id=0),
    )(A, B)


# device time: 74976 ns/iter; 1.5212x vs baseline; 1.5212x over previous
import jax
import jax.numpy as jnp
from jax import lax
from jax.experimental import pallas as pl
from jax.experimental.pallas import tpu as pltpu

N_DEV = 8
M = 1024
N = 1024
CH = M // N_DEV


def _gelu(z):
    return 0.5 * z * (1.0 + jnp.tanh(0.7978845608 * (z + 0.044715 * z * z * z)))


def kernel(A, B):
    def body(
        a_ref,
        b_ref,
        out_ref,
        z_ref,
        recv_buf,
        rs_send_sems,
        rs_recv_sems,
        ag_send_sems,
        ag_recv_sems,
    ):
        my = lax.axis_index("i")

        barrier = pltpu.get_barrier_semaphore()
        for o in range(1, N_DEV):
            pl.semaphore_signal(
                barrier,
                inc=1,
                device_id=((my + o) % N_DEV,),
                device_id_type=pl.DeviceIdType.MESH,
            )
        pl.semaphore_wait(barrier, N_DEV - 1)

        z_ref[:, :] = jnp.dot(
            a_ref[:, :], b_ref[:, :], preferred_element_type=jnp.float32
        )

        def rows(c):
            return pl.ds(c * CH, CH)

        p1 = []
        for o in range(1, N_DEV):
            e = (my + o) % N_DEV
            rdma = pltpu.make_async_remote_copy(
                src_ref=z_ref.at[rows(e)],
                dst_ref=recv_buf.at[o - 1],
                send_sem=rs_send_sems.at[o - 1],
                recv_sem=rs_recv_sems.at[o - 1],
                device_id=(e,),
                device_id_type=pl.DeviceIdType.MESH,
            )
            rdma.start()
            p1.append(rdma)
        for rdma in p1:
            rdma.wait_recv()

        acc = z_ref[rows(my), :]
        for o in range(1, N_DEV):
            acc = acc + recv_buf[o - 1]
        out_ref[rows(my), :] = _gelu(acc)

        p2 = []
        for o in range(1, N_DEV):
            e = (my + o) % N_DEV
            rdma = pltpu.make_async_remote_copy(
                src_ref=out_ref.at[rows(my)],
                dst_ref=out_ref.at[rows(my)],
                send_sem=ag_send_sems.at[o - 1],
                recv_sem=ag_recv_sems.at[o - 1],
                device_id=(e,),
                device_id_type=pl.DeviceIdType.MESH,
            )
            rdma.start()
            p2.append(rdma)
        for rdma in p2:
            rdma.wait_recv()
        for rdma in p1:
            rdma.wait_send()
        for rdma in p2:
            rdma.wait_send()

    return pl.pallas_call(
        body,
        out_shape=jax.ShapeDtypeStruct((M, N), jnp.float32),
        in_specs=[
            pl.BlockSpec(memory_space=pltpu.VMEM),
            pl.BlockSpec(memory_space=pltpu.VMEM),
        ],
        out_specs=pl.BlockSpec(memory_space=pltpu.VMEM),
        scratch_shapes=[
            pltpu.VMEM((M, N), jnp.float32),
            pltpu.VMEM((N_DEV - 1, CH, N), jnp.float32),
            pltpu.SemaphoreType.DMA((N_DEV - 1,)),
            pltpu.SemaphoreType.DMA((N_DEV - 1,)),
            pltpu.SemaphoreType.DMA((N_DEV - 1,)),
            pltpu.SemaphoreType.DMA((N_DEV - 1,)),
        ],
        compiler_params=pltpu.CompilerParams(collective_id=0),
    )(A, B)


# device time: 68043 ns/iter; 1.6762x vs baseline; 1.1019x over previous
import jax
import jax.numpy as jnp
from jax import lax
from jax.experimental import pallas as pl
from jax.experimental.pallas import tpu as pltpu

N_DEV = 8
M = 1024
N = 1024
CH = M // N_DEV
H = CH // 2
NP = N_DEV - 1


def _gelu(z):
    return 0.5 * z * (1.0 + jnp.tanh(0.7978845608 * (z + 0.044715 * z * z * z)))


def kernel(A, B):
    def body(
        a_ref,
        b_ref,
        out_ref,
        z_ref,
        recv_buf,
        rs_send_sems,
        rs_recv_sems,
        ag_send_sems,
        ag_recv_sems,
    ):
        my = lax.axis_index("i")

        barrier = pltpu.get_barrier_semaphore()
        for o in range(1, N_DEV):
            pl.semaphore_signal(
                barrier,
                inc=1,
                device_id=((my + o) % N_DEV,),
                device_id_type=pl.DeviceIdType.MESH,
            )
        pl.semaphore_wait(barrier, NP)

        def rows(c):
            return pl.ds(c * CH, CH)

        def half_rows(c, h):
            return pl.ds(c * CH + h * H, H)

        p1 = []
        for o in range(1, N_DEV):
            e = (my + o) % N_DEV
            z_ref[rows(e), :] = jnp.dot(
                a_ref[rows(e), :], b_ref[:, :],
                preferred_element_type=jnp.float32,
            )
            for h in range(2):
                rdma = pltpu.make_async_remote_copy(
                    src_ref=z_ref.at[half_rows(e, h)],
                    dst_ref=recv_buf.at[o - 1, h],
                    send_sem=rs_send_sems.at[2 * (o - 1) + h],
                    recv_sem=rs_recv_sems.at[2 * (o - 1) + h],
                    device_id=(e,),
                    device_id_type=pl.DeviceIdType.MESH,
                )
                rdma.start()
                p1.append(rdma)

        z_ref[rows(my), :] = jnp.dot(
            a_ref[rows(my), :], b_ref[:, :], preferred_element_type=jnp.float32
        )

        p2 = []
        for h in range(2):
            for o in range(1, N_DEV):
                p1[2 * (o - 1) + h].wait_recv()
            acc = z_ref[half_rows(my, h), :]
            for o in range(1, N_DEV):
                acc = acc + recv_buf[o - 1, h]
            out_ref[half_rows(my, h), :] = _gelu(acc)
            for o in range(1, N_DEV):
                e = (my + o) % N_DEV
                rdma = pltpu.make_async_remote_copy(
                    src_ref=out_ref.at[half_rows(my, h)],
                    dst_ref=out_ref.at[half_rows(my, h)],
                    send_sem=ag_send_sems.at[2 * (o - 1) + h],
                    recv_sem=ag_recv_sems.at[2 * (o - 1) + h],
                    device_id=(e,),
                    device_id_type=pl.DeviceIdType.MESH,
                )
                rdma.start()
                p2.append(rdma)

        for rdma in p2:
            rdma.wait_recv()
        for rdma in p1:
            rdma.wait_send()
        for rdma in p2:
            rdma.wait_send()

    return pl.pallas_call(
        body,
        out_shape=jax.ShapeDtypeStruct((M, N), jnp.float32),
        in_specs=[
            pl.BlockSpec(memory_space=pltpu.VMEM),
            pl.BlockSpec(memory_space=pltpu.VMEM),
        ],
        out_specs=pl.BlockSpec(memory_space=pltpu.VMEM),
        scratch_shapes=[
            pltpu.VMEM((M, N), jnp.float32),
            pltpu.VMEM((NP, 2, H, N), jnp.float32),
            pltpu.SemaphoreType.DMA((2 * NP,)),
            pltpu.SemaphoreType.DMA((2 * NP,)),
            pltpu.SemaphoreType.DMA((2 * NP,)),
            pltpu.SemaphoreType.DMA((2 * NP,)),
        ],
        compiler_params=pltpu.CompilerParams(collective_id=0),
    )(A, B)
